# baseline (speedup 1.0000x reference)
import functools

import jax
import jax.numpy as jnp
from jax import lax
from jax.experimental import pallas as pl
from jax.experimental.pallas import tpu as pltpu
from jax.experimental.pallas import tpu_sc as plsc

N = 10000
E = 320000
F_IN = 128
DIM = 95
G = 64
OUT = 10

DIMP = 96
HID2 = 192
NW = 32
CHUNK = 128
NC0 = 56
NC1 = 102
NCMAX = max(NC0, NC1)
NACC = 10112
ROWS_PER_TILE = NACC // 16

R = 1000
NBLK = N // R


def _edge_agg_kernel(x_hbm, srcp_hbm, dstp_hbm, zeros_hbm, out_hbm,
                     src_v, dst_v, rows_v, acc_sh, sem):
    c = lax.axis_index("c")
    s = lax.axis_index("s")
    wid = s * 2 + c

    pltpu.sync_copy(zeros_hbm.at[pl.ds(s * ROWS_PER_TILE, ROWS_PER_TILE)],
                    acc_sh.at[pl.ds(s * ROWS_PER_TILE, ROWS_PER_TILE)])

    pltpu.sync_copy(srcp_hbm.at[wid], src_v)
    pltpu.sync_copy(dstp_hbm.at[wid], dst_v)
    plsc.subcore_barrier()

    def body(j, carry):
        pltpu.async_copy(x_hbm.at[src_v.at[j]], rows_v, sem).wait()
        pltpu.sync_copy(rows_v, acc_sh.at[dst_v.at[j]], add=True)
        return carry

    lax.fori_loop(0, jnp.where(c == 0, NC0, NC1), body, 0)
    plsc.subcore_barrier()

    pltpu.sync_copy(acc_sh.at[pl.ds(s * ROWS_PER_TILE, ROWS_PER_TILE)],
                    out_hbm.at[c, pl.ds(s * ROWS_PER_TILE, ROWS_PER_TILE)])


def _make_edge_agg():
    mesh = plsc.VectorSubcoreMesh(core_axis_name="c", subcore_axis_name="s")
    return functools.partial(
        pl.kernel,
        mesh=mesh,
        out_type=jax.ShapeDtypeStruct((2, NACC, F_IN), jnp.float32),
        scratch_types=[
            pltpu.VMEM((NCMAX, CHUNK), jnp.int32),
            pltpu.VMEM((NCMAX, CHUNK), jnp.int32),
            pltpu.VMEM((CHUNK, F_IN), jnp.float32),
            pltpu.VMEM_SHARED((NACC, F_IN), jnp.float32),
            pltpu.SemaphoreType.DMA,
        ],
    )(_edge_agg_kernel)


_edge_agg = _make_edge_agg()


def _tc_main_body(x_ref, a0_ref, a1_ref, b_ref, W1_ref, b1_ref, W2_ref, b2_ref,
                  s1_ref, s2_ref, seg_ref, cnt_ref):
    i = pl.program_id(0)
    h = x_ref[...] + a0_ref[0] + a1_ref[0]
    t = jnp.maximum(
        jnp.dot(h, W1_ref[...], preferred_element_type=jnp.float32) + b1_ref[...],
        0.0)
    h2 = jnp.maximum(
        jnp.dot(t, W2_ref[...], preferred_element_type=jnp.float32) + b2_ref[...],
        0.0)
    oh = (b_ref[...] == lax.broadcasted_iota(jnp.int32, (R, G), 1)
          ).astype(jnp.float32)
    segp = lax.dot_general(oh, h2, (((0,), (0,)), ((), ())),
                           preferred_element_type=jnp.float32)

    @pl.when(i == 0)
    def _():
        s1_ref[...] = jnp.zeros_like(s1_ref)
        s2_ref[...] = jnp.zeros_like(s2_ref)
        seg_ref[...] = jnp.zeros_like(seg_ref)
        cnt_ref[...] = jnp.zeros_like(cnt_ref)

    s1_ref[...] += jnp.sum(h2, axis=0, keepdims=True)
    s2_ref[...] += jnp.sum(h2 * h2, axis=0, keepdims=True)
    seg_ref[...] += segp
    cnt_ref[...] += jnp.sum(oh, axis=0, keepdims=True)


def _tc_main(x, agg2, batch2, W1p, b1p, W2p, b2p):
    return pl.pallas_call(
        _tc_main_body,
        grid=(NBLK,),
        in_specs=[
            pl.BlockSpec((R, F_IN), lambda i: (i, 0)),
            pl.BlockSpec((1, R, F_IN), lambda i: (0, i, 0)),
            pl.BlockSpec((1, R, F_IN), lambda i: (1, i, 0)),
            pl.BlockSpec((R, 1), lambda i: (i, 0)),
            pl.BlockSpec((F_IN, DIMP), lambda i: (0, 0)),
            pl.BlockSpec((1, DIMP), lambda i: (0, 0)),
            pl.BlockSpec((DIMP, DIMP), lambda i: (0, 0)),
            pl.BlockSpec((1, DIMP), lambda i: (0, 0)),
        ],
        out_specs=[
            pl.BlockSpec((1, DIMP), lambda i: (0, 0)),
            pl.BlockSpec((1, DIMP), lambda i: (0, 0)),
            pl.BlockSpec((G, DIMP), lambda i: (0, 0)),
            pl.BlockSpec((1, G), lambda i: (0, 0)),
        ],
        out_shape=[
            jax.ShapeDtypeStruct((1, DIMP), jnp.float32),
            jax.ShapeDtypeStruct((1, DIMP), jnp.float32),
            jax.ShapeDtypeStruct((G, DIMP), jnp.float32),
            jax.ShapeDtypeStruct((1, G), jnp.float32),
        ],
    )(x, agg2, agg2, batch2, W1p, b1p, W2p, b2p)


def _tc_final_body(s1_ref, s2_ref, seg_ref, cnt_ref, g_ref, be_ref,
                   f1w_ref, f1b_ref, f2w_ref, f2b_ref, o_ref):
    inv_n = jnp.float32(1.0 / N)
    mean = s1_ref[...] * inv_n
    var = s2_ref[...] * inv_n - mean * mean
    a = g_ref[...] * lax.rsqrt(var + 1e-5)
    c = be_ref[...] - mean * a
    counts = jnp.reshape(cnt_ref[...], (G, 1))
    pooled = seg_ref[...] * a + counts * c
    u = jnp.maximum(
        jnp.dot(pooled, f1w_ref[...], preferred_element_type=jnp.float32)
        + f1b_ref[...], 0.0)
    o_ref[...] = (jnp.dot(u, f2w_ref[...], preferred_element_type=jnp.float32)
                  + f2b_ref[...])


def _tc_final(s1, s2, seg, cnt, gp, bp, f1wp, f1bp, f2wp, f2b2):
    return pl.pallas_call(
        _tc_final_body,
        out_shape=jax.ShapeDtypeStruct((G, OUT), jnp.float32),
    )(s1, s2, seg, cnt, gp, bp, f1wp, f1bp, f2wp, f2b2)


@jax.jit
def kernel(x, edge_index, batch, W1, b1, W2, b2, gamma, beta,
           fc1_W, fc1_b, fc2_W, fc2_b):
    src = edge_index[0]
    dst = edge_index[1]
    row_cap = NCMAX * CHUNK
    pad_dst = N + jnp.arange(row_cap, dtype=jnp.int32) % (NACC - N)
    pad_src = jnp.zeros((row_cap,), jnp.int32)
    sp, dp_, off = [], [], 0
    for w in range(NW):
        lw = (NC0 if w % 2 == 0 else NC1) * CHUNK
        take = min(lw, E - off)
        sp.append(jnp.concatenate([src[off:off + take], pad_src[:row_cap - take]]))
        dp_.append(jnp.concatenate([dst[off:off + take], pad_dst[:row_cap - take]]))
        off += take
    srcp = jnp.stack(sp).reshape(NW, NCMAX, CHUNK)
    dstp = jnp.stack(dp_).reshape(NW, NCMAX, CHUNK)
    zeros = jnp.zeros((NACC, F_IN), jnp.float32)

    agg2 = _edge_agg(x, srcp, dstp, zeros)

    dp = DIMP - DIM
    batch2 = batch.reshape(N, 1)
    W1p = jnp.pad(W1, ((0, 0), (0, dp)))
    b1p = jnp.pad(b1, (0, dp)).reshape(1, DIMP)
    W2p = jnp.pad(W2, ((0, dp), (0, dp)))
    b2p = jnp.pad(b2, (0, dp)).reshape(1, DIMP)

    s1, s2, seg, cnt = _tc_main(x, agg2, batch2, W1p, b1p, W2p, b2p)

    hp = HID2 - 2 * DIM
    gp = jnp.pad(gamma, (0, dp)).reshape(1, DIMP)
    bp = jnp.pad(beta, (0, dp)).reshape(1, DIMP)
    f1wp = jnp.pad(fc1_W, ((0, dp), (0, hp)))
    f1bp = jnp.pad(fc1_b, (0, hp)).reshape(1, HID2)
    f2wp = jnp.pad(fc2_W, ((0, hp), (0, 0)))
    f2b2 = fc2_b.reshape(1, OUT)

    return _tc_final(s1, s2, seg, cnt, gp, bp, f1wp, f1bp, f2wp, f2b2)


# speedup vs baseline: 1.2410x; 1.2410x over previous
import functools

import jax
import jax.numpy as jnp
from jax import lax
from jax.experimental import pallas as pl
from jax.experimental.pallas import tpu as pltpu
from jax.experimental.pallas import tpu_sc as plsc

N = 10000
E = 320000
F_IN = 128
DIM = 95
G = 64
OUT = 10

DIMP = 96
HID2 = 192
NW = 32
CHUNK = 128
NC0 = 102
NC1 = 56
NCMAX = max(NC0, NC1)
NACC = 10112
ROWS_PER_TILE = NACC // 16

R = 1000
NBLK = N // R


def _edge_agg_kernel(x_hbm, srcp_hbm, dstp_hbm, zeros_hbm, out_hbm,
                     src_v, dst_v, rows_v, acc_sh, sem):
    c = lax.axis_index("c")
    s = lax.axis_index("s")
    wid = s * 2 + c

    pltpu.sync_copy(zeros_hbm.at[pl.ds(s * ROWS_PER_TILE, ROWS_PER_TILE)],
                    acc_sh.at[pl.ds(s * ROWS_PER_TILE, ROWS_PER_TILE)])

    pltpu.sync_copy(srcp_hbm.at[wid], src_v)
    pltpu.sync_copy(dstp_hbm.at[wid], dst_v)
    plsc.subcore_barrier()

    def body(j, carry):
        pltpu.async_copy(x_hbm.at[src_v.at[j]], rows_v, sem).wait()
        pltpu.sync_copy(rows_v, acc_sh.at[dst_v.at[j]], add=True)
        return carry

    lax.fori_loop(0, jnp.where(c == 0, NC0, NC1), body, 0)
    plsc.subcore_barrier()

    pltpu.sync_copy(acc_sh.at[pl.ds(s * ROWS_PER_TILE, ROWS_PER_TILE)],
                    out_hbm.at[c, pl.ds(s * ROWS_PER_TILE, ROWS_PER_TILE)])


def _make_edge_agg():
    mesh = plsc.VectorSubcoreMesh(core_axis_name="c", subcore_axis_name="s")
    return functools.partial(
        pl.kernel,
        mesh=mesh,
        out_type=jax.ShapeDtypeStruct((2, NACC, F_IN), jnp.float32),
        scratch_types=[
            pltpu.VMEM((NCMAX, CHUNK), jnp.int32),
            pltpu.VMEM((NCMAX, CHUNK), jnp.int32),
            pltpu.VMEM((CHUNK, F_IN), jnp.float32),
            pltpu.VMEM_SHARED((NACC, F_IN), jnp.float32),
            pltpu.SemaphoreType.DMA,
        ],
    )(_edge_agg_kernel)


_edge_agg = _make_edge_agg()


def _tc_main_body(x_ref, a0_ref, a1_ref, b_ref, W1_ref, b1_ref, W2_ref, b2_ref,
                  s1_ref, s2_ref, seg_ref, cnt_ref):
    i = pl.program_id(0)
    h = x_ref[...] + a0_ref[0] + a1_ref[0]
    t = jnp.maximum(
        jnp.dot(h, W1_ref[...], preferred_element_type=jnp.float32) + b1_ref[...],
        0.0)
    h2 = jnp.maximum(
        jnp.dot(t, W2_ref[...], preferred_element_type=jnp.float32) + b2_ref[...],
        0.0)
    oh = (b_ref[...] == lax.broadcasted_iota(jnp.int32, (R, G), 1)
          ).astype(jnp.float32)
    segp = lax.dot_general(oh, h2, (((0,), (0,)), ((), ())),
                           preferred_element_type=jnp.float32)

    @pl.when(i == 0)
    def _():
        s1_ref[...] = jnp.zeros_like(s1_ref)
        s2_ref[...] = jnp.zeros_like(s2_ref)
        seg_ref[...] = jnp.zeros_like(seg_ref)
        cnt_ref[...] = jnp.zeros_like(cnt_ref)

    s1_ref[...] += jnp.sum(h2, axis=0, keepdims=True)
    s2_ref[...] += jnp.sum(h2 * h2, axis=0, keepdims=True)
    seg_ref[...] += segp
    cnt_ref[...] += jnp.sum(oh, axis=0, keepdims=True)


def _tc_main(x, agg2, batch2, W1p, b1p, W2p, b2p):
    return pl.pallas_call(
        _tc_main_body,
        grid=(NBLK,),
        in_specs=[
            pl.BlockSpec((R, F_IN), lambda i: (i, 0)),
            pl.BlockSpec((1, R, F_IN), lambda i: (0, i, 0)),
            pl.BlockSpec((1, R, F_IN), lambda i: (1, i, 0)),
            pl.BlockSpec((R, 1), lambda i: (i, 0)),
            pl.BlockSpec((F_IN, DIMP), lambda i: (0, 0)),
            pl.BlockSpec((1, DIMP), lambda i: (0, 0)),
            pl.BlockSpec((DIMP, DIMP), lambda i: (0, 0)),
            pl.BlockSpec((1, DIMP), lambda i: (0, 0)),
        ],
        out_specs=[
            pl.BlockSpec((1, DIMP), lambda i: (0, 0)),
            pl.BlockSpec((1, DIMP), lambda i: (0, 0)),
            pl.BlockSpec((G, DIMP), lambda i: (0, 0)),
            pl.BlockSpec((1, G), lambda i: (0, 0)),
        ],
        out_shape=[
            jax.ShapeDtypeStruct((1, DIMP), jnp.float32),
            jax.ShapeDtypeStruct((1, DIMP), jnp.float32),
            jax.ShapeDtypeStruct((G, DIMP), jnp.float32),
            jax.ShapeDtypeStruct((1, G), jnp.float32),
        ],
    )(x, agg2, agg2, batch2, W1p, b1p, W2p, b2p)


def _tc_final_body(s1_ref, s2_ref, seg_ref, cnt_ref, g_ref, be_ref,
                   f1w_ref, f1b_ref, f2w_ref, f2b_ref, o_ref):
    inv_n = jnp.float32(1.0 / N)
    mean = s1_ref[...] * inv_n
    var = s2_ref[...] * inv_n - mean * mean
    a = g_ref[...] * lax.rsqrt(var + 1e-5)
    c = be_ref[...] - mean * a
    counts = jnp.reshape(cnt_ref[...], (G, 1))
    pooled = seg_ref[...] * a + counts * c
    u = jnp.maximum(
        jnp.dot(pooled, f1w_ref[...], preferred_element_type=jnp.float32)
        + f1b_ref[...], 0.0)
    o_ref[...] = (jnp.dot(u, f2w_ref[...], preferred_element_type=jnp.float32)
                  + f2b_ref[...])


def _tc_final(s1, s2, seg, cnt, gp, bp, f1wp, f1bp, f2wp, f2b2):
    return pl.pallas_call(
        _tc_final_body,
        out_shape=jax.ShapeDtypeStruct((G, OUT), jnp.float32),
    )(s1, s2, seg, cnt, gp, bp, f1wp, f1bp, f2wp, f2b2)


@jax.jit
def kernel(x, edge_index, batch, W1, b1, W2, b2, gamma, beta,
           fc1_W, fc1_b, fc2_W, fc2_b):
    src = edge_index[0]
    dst = edge_index[1]
    row_cap = NCMAX * CHUNK
    pad_dst = N + jnp.arange(row_cap, dtype=jnp.int32) % (NACC - N)
    pad_src = jnp.zeros((row_cap,), jnp.int32)
    sp, dp_, off = [], [], 0
    for w in range(NW):
        lw = (NC0 if w % 2 == 0 else NC1) * CHUNK
        take = min(lw, E - off)
        sp.append(jnp.concatenate([src[off:off + take], pad_src[:row_cap - take]]))
        dp_.append(jnp.concatenate([dst[off:off + take], pad_dst[:row_cap - take]]))
        off += take
    srcp = jnp.stack(sp).reshape(NW, NCMAX, CHUNK)
    dstp = jnp.stack(dp_).reshape(NW, NCMAX, CHUNK)
    zeros = jnp.zeros((NACC, F_IN), jnp.float32)

    agg2 = _edge_agg(x, srcp, dstp, zeros)

    dp = DIMP - DIM
    batch2 = batch.reshape(N, 1)
    W1p = jnp.pad(W1, ((0, 0), (0, dp)))
    b1p = jnp.pad(b1, (0, dp)).reshape(1, DIMP)
    W2p = jnp.pad(W2, ((0, dp), (0, dp)))
    b2p = jnp.pad(b2, (0, dp)).reshape(1, DIMP)

    s1, s2, seg, cnt = _tc_main(x, agg2, batch2, W1p, b1p, W2p, b2p)

    hp = HID2 - 2 * DIM
    gp = jnp.pad(gamma, (0, dp)).reshape(1, DIMP)
    bp = jnp.pad(beta, (0, dp)).reshape(1, DIMP)
    f1wp = jnp.pad(fc1_W, ((0, dp), (0, hp)))
    f1bp = jnp.pad(fc1_b, (0, hp)).reshape(1, HID2)
    f2wp = jnp.pad(fc2_W, ((0, hp), (0, 0)))
    f2b2 = fc2_b.reshape(1, OUT)

    return _tc_final(s1, s2, seg, cnt, gp, bp, f1wp, f1bp, f2wp, f2b2)


# speedup vs baseline: 1.2871x; 1.0371x over previous
import functools

import jax
import jax.numpy as jnp
from jax import lax
from jax.experimental import pallas as pl
from jax.experimental.pallas import tpu as pltpu
from jax.experimental.pallas import tpu_sc as plsc

N = 10000
E = 320000
F_IN = 128
DIM = 95
G = 64
OUT = 10

DIMP = 96
HID2 = 192
NW = 32
CHUNK = 128
NC0 = 110
NC1 = 48
NCMAX = max(NC0, NC1)
NACC = 10112
ROWS_PER_TILE = NACC // 16

R = 1000
NBLK = N // R


def _edge_agg_kernel(x_hbm, srcp_hbm, dstp_hbm, zeros_hbm, out_hbm,
                     src_v, dst_v, rows_v, acc_sh, sem):
    c = lax.axis_index("c")
    s = lax.axis_index("s")
    wid = s * 2 + c

    pltpu.sync_copy(zeros_hbm.at[pl.ds(s * ROWS_PER_TILE, ROWS_PER_TILE)],
                    acc_sh.at[pl.ds(s * ROWS_PER_TILE, ROWS_PER_TILE)])

    pltpu.sync_copy(srcp_hbm.at[wid], src_v)
    pltpu.sync_copy(dstp_hbm.at[wid], dst_v)
    plsc.subcore_barrier()

    def body(j, carry):
        pltpu.async_copy(x_hbm.at[src_v.at[j]], rows_v, sem).wait()
        pltpu.sync_copy(rows_v, acc_sh.at[dst_v.at[j]], add=True)
        return carry

    lax.fori_loop(0, jnp.where(c == 0, NC0, NC1), body, 0)
    plsc.subcore_barrier()

    pltpu.sync_copy(acc_sh.at[pl.ds(s * ROWS_PER_TILE, ROWS_PER_TILE)],
                    out_hbm.at[c, pl.ds(s * ROWS_PER_TILE, ROWS_PER_TILE)])


def _make_edge_agg():
    mesh = plsc.VectorSubcoreMesh(core_axis_name="c", subcore_axis_name="s")
    return functools.partial(
        pl.kernel,
        mesh=mesh,
        out_type=jax.ShapeDtypeStruct((2, NACC, F_IN), jnp.float32),
        scratch_types=[
            pltpu.VMEM((NCMAX, CHUNK), jnp.int32),
            pltpu.VMEM((NCMAX, CHUNK), jnp.int32),
            pltpu.VMEM((CHUNK, F_IN), jnp.float32),
            pltpu.VMEM_SHARED((NACC, F_IN), jnp.float32),
            pltpu.SemaphoreType.DMA,
        ],
    )(_edge_agg_kernel)


_edge_agg = _make_edge_agg()


def _tc_main_body(x_ref, a0_ref, a1_ref, b_ref, W1_ref, b1_ref, W2_ref, b2_ref,
                  s1_ref, s2_ref, seg_ref, cnt_ref):
    i = pl.program_id(0)
    h = x_ref[...] + a0_ref[0] + a1_ref[0]
    t = jnp.maximum(
        jnp.dot(h, W1_ref[...], preferred_element_type=jnp.float32) + b1_ref[...],
        0.0)
    h2 = jnp.maximum(
        jnp.dot(t, W2_ref[...], preferred_element_type=jnp.float32) + b2_ref[...],
        0.0)
    oh = (b_ref[...] == lax.broadcasted_iota(jnp.int32, (R, G), 1)
          ).astype(jnp.float32)
    segp = lax.dot_general(oh, h2, (((0,), (0,)), ((), ())),
                           preferred_element_type=jnp.float32)

    @pl.when(i == 0)
    def _():
        s1_ref[...] = jnp.zeros_like(s1_ref)
        s2_ref[...] = jnp.zeros_like(s2_ref)
        seg_ref[...] = jnp.zeros_like(seg_ref)
        cnt_ref[...] = jnp.zeros_like(cnt_ref)

    s1_ref[...] += jnp.sum(h2, axis=0, keepdims=True)
    s2_ref[...] += jnp.sum(h2 * h2, axis=0, keepdims=True)
    seg_ref[...] += segp
    cnt_ref[...] += jnp.sum(oh, axis=0, keepdims=True)


def _tc_main(x, agg2, batch2, W1p, b1p, W2p, b2p):
    return pl.pallas_call(
        _tc_main_body,
        grid=(NBLK,),
        in_specs=[
            pl.BlockSpec((R, F_IN), lambda i: (i, 0)),
            pl.BlockSpec((1, R, F_IN), lambda i: (0, i, 0)),
            pl.BlockSpec((1, R, F_IN), lambda i: (1, i, 0)),
            pl.BlockSpec((R, 1), lambda i: (i, 0)),
            pl.BlockSpec((F_IN, DIMP), lambda i: (0, 0)),
            pl.BlockSpec((1, DIMP), lambda i: (0, 0)),
            pl.BlockSpec((DIMP, DIMP), lambda i: (0, 0)),
            pl.BlockSpec((1, DIMP), lambda i: (0, 0)),
        ],
        out_specs=[
            pl.BlockSpec((1, DIMP), lambda i: (0, 0)),
            pl.BlockSpec((1, DIMP), lambda i: (0, 0)),
            pl.BlockSpec((G, DIMP), lambda i: (0, 0)),
            pl.BlockSpec((1, G), lambda i: (0, 0)),
        ],
        out_shape=[
            jax.ShapeDtypeStruct((1, DIMP), jnp.float32),
            jax.ShapeDtypeStruct((1, DIMP), jnp.float32),
            jax.ShapeDtypeStruct((G, DIMP), jnp.float32),
            jax.ShapeDtypeStruct((1, G), jnp.float32),
        ],
    )(x, agg2, agg2, batch2, W1p, b1p, W2p, b2p)


def _tc_final_body(s1_ref, s2_ref, seg_ref, cnt_ref, g_ref, be_ref,
                   f1w_ref, f1b_ref, f2w_ref, f2b_ref, o_ref):
    inv_n = jnp.float32(1.0 / N)
    mean = s1_ref[...] * inv_n
    var = s2_ref[...] * inv_n - mean * mean
    a = g_ref[...] * lax.rsqrt(var + 1e-5)
    c = be_ref[...] - mean * a
    counts = jnp.reshape(cnt_ref[...], (G, 1))
    pooled = seg_ref[...] * a + counts * c
    u = jnp.maximum(
        jnp.dot(pooled, f1w_ref[...], preferred_element_type=jnp.float32)
        + f1b_ref[...], 0.0)
    o_ref[...] = (jnp.dot(u, f2w_ref[...], preferred_element_type=jnp.float32)
                  + f2b_ref[...])


def _tc_final(s1, s2, seg, cnt, gp, bp, f1wp, f1bp, f2wp, f2b2):
    return pl.pallas_call(
        _tc_final_body,
        out_shape=jax.ShapeDtypeStruct((G, OUT), jnp.float32),
    )(s1, s2, seg, cnt, gp, bp, f1wp, f1bp, f2wp, f2b2)


@jax.jit
def kernel(x, edge_index, batch, W1, b1, W2, b2, gamma, beta,
           fc1_W, fc1_b, fc2_W, fc2_b):
    src = edge_index[0]
    dst = edge_index[1]
    row_cap = NCMAX * CHUNK
    pad_dst = N + jnp.arange(row_cap, dtype=jnp.int32) % (NACC - N)
    pad_src = jnp.zeros((row_cap,), jnp.int32)
    sp, dp_, off = [], [], 0
    for w in range(NW):
        lw = (NC0 if w % 2 == 0 else NC1) * CHUNK
        take = min(lw, E - off)
        sp.append(jnp.concatenate([src[off:off + take], pad_src[:row_cap - take]]))
        dp_.append(jnp.concatenate([dst[off:off + take], pad_dst[:row_cap - take]]))
        off += take
    srcp = jnp.stack(sp).reshape(NW, NCMAX, CHUNK)
    dstp = jnp.stack(dp_).reshape(NW, NCMAX, CHUNK)
    zeros = jnp.zeros((NACC, F_IN), jnp.float32)

    agg2 = _edge_agg(x, srcp, dstp, zeros)

    dp = DIMP - DIM
    batch2 = batch.reshape(N, 1)
    W1p = jnp.pad(W1, ((0, 0), (0, dp)))
    b1p = jnp.pad(b1, (0, dp)).reshape(1, DIMP)
    W2p = jnp.pad(W2, ((0, dp), (0, dp)))
    b2p = jnp.pad(b2, (0, dp)).reshape(1, DIMP)

    s1, s2, seg, cnt = _tc_main(x, agg2, batch2, W1p, b1p, W2p, b2p)

    hp = HID2 - 2 * DIM
    gp = jnp.pad(gamma, (0, dp)).reshape(1, DIMP)
    bp = jnp.pad(beta, (0, dp)).reshape(1, DIMP)
    f1wp = jnp.pad(fc1_W, ((0, dp), (0, hp)))
    f1bp = jnp.pad(fc1_b, (0, hp)).reshape(1, HID2)
    f2wp = jnp.pad(fc2_W, ((0, hp), (0, 0)))
    f2b2 = fc2_b.reshape(1, OUT)

    return _tc_final(s1, s2, seg, cnt, gp, bp, f1wp, f1bp, f2wp, f2b2)


# speedup vs baseline: 1.3359x; 1.0379x over previous
import functools

import jax
import jax.numpy as jnp
from jax import lax
from jax.experimental import pallas as pl
from jax.experimental.pallas import tpu as pltpu
from jax.experimental.pallas import tpu_sc as plsc

N = 10000
E = 320000
F_IN = 128
DIM = 95
G = 64
OUT = 10

DIMP = 96
HID2 = 192
NW = 32
CHUNK = 128
NC0 = 116
NC1 = 42
NCMAX = max(NC0, NC1)
NACC = 10112
ROWS_PER_TILE = NACC // 16

R = 1000
NBLK = N // R


def _edge_agg_kernel(x_hbm, srcp_hbm, dstp_hbm, zeros_hbm, out_hbm,
                     src_v, dst_v, rows_v, acc_sh, sem):
    c = lax.axis_index("c")
    s = lax.axis_index("s")
    wid = s * 2 + c

    pltpu.sync_copy(zeros_hbm.at[pl.ds(s * ROWS_PER_TILE, ROWS_PER_TILE)],
                    acc_sh.at[pl.ds(s * ROWS_PER_TILE, ROWS_PER_TILE)])

    pltpu.sync_copy(srcp_hbm.at[wid], src_v)
    pltpu.sync_copy(dstp_hbm.at[wid], dst_v)
    plsc.subcore_barrier()

    def body(j, carry):
        pltpu.async_copy(x_hbm.at[src_v.at[j]], rows_v, sem).wait()
        pltpu.sync_copy(rows_v, acc_sh.at[dst_v.at[j]], add=True)
        return carry

    lax.fori_loop(0, jnp.where(c == 0, NC0, NC1), body, 0)
    plsc.subcore_barrier()

    pltpu.sync_copy(acc_sh.at[pl.ds(s * ROWS_PER_TILE, ROWS_PER_TILE)],
                    out_hbm.at[c, pl.ds(s * ROWS_PER_TILE, ROWS_PER_TILE)])


def _make_edge_agg():
    mesh = plsc.VectorSubcoreMesh(core_axis_name="c", subcore_axis_name="s")
    return functools.partial(
        pl.kernel,
        mesh=mesh,
        out_type=jax.ShapeDtypeStruct((2, NACC, F_IN), jnp.float32),
        scratch_types=[
            pltpu.VMEM((NCMAX, CHUNK), jnp.int32),
            pltpu.VMEM((NCMAX, CHUNK), jnp.int32),
            pltpu.VMEM((CHUNK, F_IN), jnp.float32),
            pltpu.VMEM_SHARED((NACC, F_IN), jnp.float32),
            pltpu.SemaphoreType.DMA,
        ],
    )(_edge_agg_kernel)


_edge_agg = _make_edge_agg()


def _tc_main_body(x_ref, a0_ref, a1_ref, b_ref, W1_ref, b1_ref, W2_ref, b2_ref,
                  s1_ref, s2_ref, seg_ref, cnt_ref):
    i = pl.program_id(0)
    h = x_ref[...] + a0_ref[0] + a1_ref[0]
    t = jnp.maximum(
        jnp.dot(h, W1_ref[...], preferred_element_type=jnp.float32) + b1_ref[...],
        0.0)
    h2 = jnp.maximum(
        jnp.dot(t, W2_ref[...], preferred_element_type=jnp.float32) + b2_ref[...],
        0.0)
    oh = (b_ref[...] == lax.broadcasted_iota(jnp.int32, (R, G), 1)
          ).astype(jnp.float32)
    segp = lax.dot_general(oh, h2, (((0,), (0,)), ((), ())),
                           preferred_element_type=jnp.float32)

    @pl.when(i == 0)
    def _():
        s1_ref[...] = jnp.zeros_like(s1_ref)
        s2_ref[...] = jnp.zeros_like(s2_ref)
        seg_ref[...] = jnp.zeros_like(seg_ref)
        cnt_ref[...] = jnp.zeros_like(cnt_ref)

    s1_ref[...] += jnp.sum(h2, axis=0, keepdims=True)
    s2_ref[...] += jnp.sum(h2 * h2, axis=0, keepdims=True)
    seg_ref[...] += segp
    cnt_ref[...] += jnp.sum(oh, axis=0, keepdims=True)


def _tc_main(x, agg2, batch2, W1p, b1p, W2p, b2p):
    return pl.pallas_call(
        _tc_main_body,
        grid=(NBLK,),
        in_specs=[
            pl.BlockSpec((R, F_IN), lambda i: (i, 0)),
            pl.BlockSpec((1, R, F_IN), lambda i: (0, i, 0)),
            pl.BlockSpec((1, R, F_IN), lambda i: (1, i, 0)),
            pl.BlockSpec((R, 1), lambda i: (i, 0)),
            pl.BlockSpec((F_IN, DIMP), lambda i: (0, 0)),
            pl.BlockSpec((1, DIMP), lambda i: (0, 0)),
            pl.BlockSpec((DIMP, DIMP), lambda i: (0, 0)),
            pl.BlockSpec((1, DIMP), lambda i: (0, 0)),
        ],
        out_specs=[
            pl.BlockSpec((1, DIMP), lambda i: (0, 0)),
            pl.BlockSpec((1, DIMP), lambda i: (0, 0)),
            pl.BlockSpec((G, DIMP), lambda i: (0, 0)),
            pl.BlockSpec((1, G), lambda i: (0, 0)),
        ],
        out_shape=[
            jax.ShapeDtypeStruct((1, DIMP), jnp.float32),
            jax.ShapeDtypeStruct((1, DIMP), jnp.float32),
            jax.ShapeDtypeStruct((G, DIMP), jnp.float32),
            jax.ShapeDtypeStruct((1, G), jnp.float32),
        ],
    )(x, agg2, agg2, batch2, W1p, b1p, W2p, b2p)


def _tc_final_body(s1_ref, s2_ref, seg_ref, cnt_ref, g_ref, be_ref,
                   f1w_ref, f1b_ref, f2w_ref, f2b_ref, o_ref):
    inv_n = jnp.float32(1.0 / N)
    mean = s1_ref[...] * inv_n
    var = s2_ref[...] * inv_n - mean * mean
    a = g_ref[...] * lax.rsqrt(var + 1e-5)
    c = be_ref[...] - mean * a
    counts = jnp.reshape(cnt_ref[...], (G, 1))
    pooled = seg_ref[...] * a + counts * c
    u = jnp.maximum(
        jnp.dot(pooled, f1w_ref[...], preferred_element_type=jnp.float32)
        + f1b_ref[...], 0.0)
    o_ref[...] = (jnp.dot(u, f2w_ref[...], preferred_element_type=jnp.float32)
                  + f2b_ref[...])


def _tc_final(s1, s2, seg, cnt, gp, bp, f1wp, f1bp, f2wp, f2b2):
    return pl.pallas_call(
        _tc_final_body,
        out_shape=jax.ShapeDtypeStruct((G, OUT), jnp.float32),
    )(s1, s2, seg, cnt, gp, bp, f1wp, f1bp, f2wp, f2b2)


@jax.jit
def kernel(x, edge_index, batch, W1, b1, W2, b2, gamma, beta,
           fc1_W, fc1_b, fc2_W, fc2_b):
    src = edge_index[0]
    dst = edge_index[1]
    row_cap = NCMAX * CHUNK
    pad_dst = N + jnp.arange(row_cap, dtype=jnp.int32) % (NACC - N)
    pad_src = jnp.zeros((row_cap,), jnp.int32)
    sp, dp_, off = [], [], 0
    for w in range(NW):
        lw = (NC0 if w % 2 == 0 else NC1) * CHUNK
        take = min(lw, E - off)
        sp.append(jnp.concatenate([src[off:off + take], pad_src[:row_cap - take]]))
        dp_.append(jnp.concatenate([dst[off:off + take], pad_dst[:row_cap - take]]))
        off += take
    srcp = jnp.stack(sp).reshape(NW, NCMAX, CHUNK)
    dstp = jnp.stack(dp_).reshape(NW, NCMAX, CHUNK)
    zeros = jnp.zeros((NACC, F_IN), jnp.float32)

    agg2 = _edge_agg(x, srcp, dstp, zeros)

    dp = DIMP - DIM
    batch2 = batch.reshape(N, 1)
    W1p = jnp.pad(W1, ((0, 0), (0, dp)))
    b1p = jnp.pad(b1, (0, dp)).reshape(1, DIMP)
    W2p = jnp.pad(W2, ((0, dp), (0, dp)))
    b2p = jnp.pad(b2, (0, dp)).reshape(1, DIMP)

    s1, s2, seg, cnt = _tc_main(x, agg2, batch2, W1p, b1p, W2p, b2p)

    hp = HID2 - 2 * DIM
    gp = jnp.pad(gamma, (0, dp)).reshape(1, DIMP)
    bp = jnp.pad(beta, (0, dp)).reshape(1, DIMP)
    f1wp = jnp.pad(fc1_W, ((0, dp), (0, hp)))
    f1bp = jnp.pad(fc1_b, (0, hp)).reshape(1, HID2)
    f2wp = jnp.pad(fc2_W, ((0, hp), (0, 0)))
    f2b2 = fc2_b.reshape(1, OUT)

    return _tc_final(s1, s2, seg, cnt, gp, bp, f1wp, f1bp, f2wp, f2b2)
